# baseline (speedup 1.0000x reference)
import functools

import jax
import jax.numpy as jnp
from jax import lax
from jax.experimental import pallas as pl
from jax.experimental.pallas import tpu as pltpu
from jax.experimental.pallas import tpu_sc as plsc

B, N, K, H = 32, 1024, 16, 128
HH = H // 2
NC, NS = 2, 16
CHUNK = 128
L = 16


PLANE = N * HH
FU = 4


def _sc_agg(uT, w1, adr):
    mesh = plsc.VectorSubcoreMesh(core_axis_name="c", subcore_axis_name="s")

    @functools.partial(
        pl.kernel,
        out_type=jax.ShapeDtypeStruct((2 * B, HH, N), jnp.float32),
        mesh=mesh,
        compiler_params=pltpu.CompilerParams(needs_layout_passes=False),
        scratch_types=[
            pltpu.VMEM((PLANE,), jnp.float32),
            pltpu.VMEM((K, CHUNK), jnp.int32),
            pltpu.VMEM((HH, CHUNK), jnp.float32),
            pltpu.VMEM((HH, CHUNK), jnp.float32),
        ],
    )
    def body(u_hbm, w_hbm, adr_hbm, agg_hbm, w_v, adr_v, u_v, a_v):
        g = lax.axis_index("s") * NC + lax.axis_index("c")
        for h in range(2):
            item = 2 * g + h
            pltpu.sync_copy(w_hbm.at[pl.ds(item * PLANE, PLANE)], w_v)

            def chunk_body(ci, _):
                pltpu.sync_copy(adr_hbm.at[g, :, pl.ds(ci * CHUNK, CHUNK)],
                                adr_v)
                pltpu.sync_copy(u_hbm.at[item, :, pl.ds(ci * CHUNK, CHUNK)],
                                u_v)

                def group_body(gi, _):
                    jvs = [adr_v[k, pl.ds(gi * L, L)] for k in range(K)]

                    def f_body(fi, _):
                        for ff in range(FU):
                            f = fi * FU + ff
                            uf = u_v[f, pl.ds(gi * L, L)]
                            acc0 = jnp.zeros((L,), jnp.float32)
                            acc1 = jnp.zeros((L,), jnp.float32)
                            ws = w_v.at[pl.ds(pl.multiple_of(f * N, N), N)]
                            for k in range(0, K, 2):
                                g0 = plsc.load_gather(ws, [jvs[k]])
                                g1 = plsc.load_gather(ws, [jvs[k + 1]])
                                acc0 = acc0 + jnp.maximum(uf + g0, 0.0)
                                acc1 = acc1 + jnp.maximum(uf + g1, 0.0)
                            a_v[f, pl.ds(gi * L, L)] = acc0 + acc1
                        return 0

                    lax.fori_loop(0, HH // FU, f_body, 0)
                    return 0

                lax.fori_loop(0, CHUNK // L, group_body, 0)
                pltpu.sync_copy(a_v,
                                agg_hbm.at[item, :, pl.ds(ci * CHUNK, CHUNK)])
                return 0

            lax.fori_loop(0, N // CHUNK, chunk_body, 0)

    return body(uT, w1, adr)


def _ln(x, g, b):
    m = x.mean(-1, keepdims=True)
    v = x.var(-1, keepdims=True)
    return g * (x - m) / jnp.sqrt(v + 1e-5) + b


def _bn(x, g, b):
    m = x.mean(0)
    v = x.var(0)
    return g * (x - m) / jnp.sqrt(v + 1e-5) + b


def kernel(node_feat, pos, mask, scalar_feat, params, edge_idx):
    p = params
    x = jax.nn.relu(_ln(node_feat @ p['emb_W'] + p['emb_b'],
                        p['emb_g'], p['emb_be']))
    adr = edge_idx.astype(jnp.int32).transpose(0, 2, 1)

    def to_uT(a):
        return (a.reshape(B, N, 2, HH).transpose(0, 2, 3, 1)
                .reshape(2 * B, HH, N))

    def to_w1(a):
        return to_uT(a).reshape(-1)

    def from_aggT(a):
        return (a.reshape(B, 2, HH, N).transpose(0, 3, 1, 2)
                .reshape(B, N, H))

    for lp in p['mp']:
        eW = lp['eW']
        q = pos @ eW[2 * H:]
        u = (x @ eW[:H] - q + lp['eb']) * (1.0 / K)
        w = (x @ eW[H:2 * H] + q) * (1.0 / K)
        agg = from_aggT(_sc_agg(to_uT(u), to_w1(w), adr))
        nW = lp['nW']
        upd = jax.nn.relu(_ln(x @ nW[:H] + agg @ nW[H:] + lp['nb'],
                              lp['ng'], lp['nbe']))
        x = upd * mask[:, :, None]
    ms = jnp.clip(mask.sum(axis=1, keepdims=True), 1, None)
    gf = (x * mask[:, :, None]).sum(axis=1) / ms
    gf = jax.nn.relu(gf @ p['ro_W'] + p['ro_b'])
    s = jax.nn.relu(_bn(scalar_feat @ p['s1_W'] + p['s1_b'],
                        p['s1_g'], p['s1_be']))
    s = jax.nn.relu(_bn(s @ p['s2_W'] + p['s2_b'], p['s2_g'], p['s2_be']))
    c = jnp.concatenate([gf, s], axis=-1)
    h = jax.nn.relu(_bn(c @ p['h1_W'] + p['h1_b'], p['h1_g'], p['h1_be']))
    return h @ p['h2_W'] + p['h2_b']


# speedup vs baseline: 1.1014x; 1.1014x over previous
import functools

import jax
import jax.numpy as jnp
from jax import lax
from jax.experimental import pallas as pl
from jax.experimental.pallas import tpu as pltpu
from jax.experimental.pallas import tpu_sc as plsc

B, N, K, H = 32, 1024, 16, 128
HH = H // 2
NC, NS = 2, 16
CHUNK = 256
L = 16


PLANE = N * HH
FU = 2


def _sc_agg(uT, w1, adr):
    mesh = plsc.VectorSubcoreMesh(core_axis_name="c", subcore_axis_name="s")

    @functools.partial(
        pl.kernel,
        out_type=jax.ShapeDtypeStruct((2 * B, HH, N), jnp.float32),
        mesh=mesh,
        compiler_params=pltpu.CompilerParams(needs_layout_passes=False),
        scratch_types=[
            pltpu.VMEM((PLANE,), jnp.float32),
            pltpu.VMEM((K, CHUNK), jnp.int32),
            pltpu.VMEM((HH, CHUNK), jnp.float32),
            pltpu.VMEM((HH, CHUNK), jnp.float32),
        ],
    )
    def body(u_hbm, w_hbm, adr_hbm, agg_hbm, w_v, adr_v, u_v, a_v):
        g = lax.axis_index("s") * NC + lax.axis_index("c")
        for h in range(2):
            item = 2 * g + h
            pltpu.sync_copy(w_hbm.at[pl.ds(item * PLANE, PLANE)], w_v)

            def chunk_body(ci, _):
                pltpu.sync_copy(adr_hbm.at[g, :, pl.ds(ci * CHUNK, CHUNK)],
                                adr_v)
                pltpu.sync_copy(u_hbm.at[item, :, pl.ds(ci * CHUNK, CHUNK)],
                                u_v)

                def group_body(gi, _):
                    jvs = [adr_v[k, pl.ds(gi * L, L)] for k in range(K)]

                    def f_body(fi, _):
                        for ff in range(FU):
                            f = fi * FU + ff
                            uf = u_v[f, pl.ds(gi * L, L)]
                            acc0 = jnp.zeros((L,), jnp.float32)
                            acc1 = jnp.zeros((L,), jnp.float32)
                            ws = w_v.at[pl.ds(pl.multiple_of(f * N, N), N)]
                            for k in range(0, K, 2):
                                g0 = plsc.load_gather(ws, [jvs[k]])
                                g1 = plsc.load_gather(ws, [jvs[k + 1]])
                                acc0 = acc0 + jnp.maximum(uf + g0, 0.0)
                                acc1 = acc1 + jnp.maximum(uf + g1, 0.0)
                            a_v[f, pl.ds(gi * L, L)] = acc0 + acc1
                        return 0

                    lax.fori_loop(0, HH // FU, f_body, 0)
                    return 0

                lax.fori_loop(0, CHUNK // L, group_body, 0)
                pltpu.sync_copy(a_v,
                                agg_hbm.at[item, :, pl.ds(ci * CHUNK, CHUNK)])
                return 0

            lax.fori_loop(0, N // CHUNK, chunk_body, 0)

    return body(uT, w1, adr)


def _ln(x, g, b):
    m = x.mean(-1, keepdims=True)
    v = x.var(-1, keepdims=True)
    return g * (x - m) / jnp.sqrt(v + 1e-5) + b


def _bn(x, g, b):
    m = x.mean(0)
    v = x.var(0)
    return g * (x - m) / jnp.sqrt(v + 1e-5) + b


def kernel(node_feat, pos, mask, scalar_feat, params, edge_idx):
    p = params
    x = jax.nn.relu(_ln(node_feat @ p['emb_W'] + p['emb_b'],
                        p['emb_g'], p['emb_be']))
    adr = edge_idx.astype(jnp.int32).transpose(0, 2, 1)

    def to_uT(a):
        return (a.reshape(B, N, 2, HH).transpose(0, 2, 3, 1)
                .reshape(2 * B, HH, N))

    def to_w1(a):
        return to_uT(a).reshape(-1)

    def from_aggT(a):
        return (a.reshape(B, 2, HH, N).transpose(0, 3, 1, 2)
                .reshape(B, N, H))

    for lp in p['mp']:
        eW = lp['eW']
        q = pos @ eW[2 * H:]
        u = (x @ eW[:H] - q + lp['eb']) * (1.0 / K)
        w = (x @ eW[H:2 * H] + q) * (1.0 / K)
        agg = from_aggT(_sc_agg(to_uT(u), to_w1(w), adr))
        nW = lp['nW']
        upd = jax.nn.relu(_ln(x @ nW[:H] + agg @ nW[H:] + lp['nb'],
                              lp['ng'], lp['nbe']))
        x = upd * mask[:, :, None]
    ms = jnp.clip(mask.sum(axis=1, keepdims=True), 1, None)
    gf = (x * mask[:, :, None]).sum(axis=1) / ms
    gf = jax.nn.relu(gf @ p['ro_W'] + p['ro_b'])
    s = jax.nn.relu(_bn(scalar_feat @ p['s1_W'] + p['s1_b'],
                        p['s1_g'], p['s1_be']))
    s = jax.nn.relu(_bn(s @ p['s2_W'] + p['s2_b'], p['s2_g'], p['s2_be']))
    c = jnp.concatenate([gf, s], axis=-1)
    h = jax.nn.relu(_bn(c @ p['h1_W'] + p['h1_b'], p['h1_g'], p['h1_be']))
    return h @ p['h2_W'] + p['h2_b']
